# baseline (speedup 1.0000x reference)
import jax
import jax.numpy as jnp
from jax.experimental import pallas as pl
from jax.experimental.pallas import tpu as pltpu


def _make_lstm_body(num_layers, seq_len, b_pad, d_pad, hid, n_streams):
    bs = b_pad // n_streams
    kk = hid + d_pad

    def body(x_ref, whh_ref, wih_ref, b_ref,
             hid_ref, cell_ref,
             seq_ref, wcat_ref):
        sc = jnp.where(
            jax.lax.broadcasted_iota(jnp.int32, (1, 4 * hid), 1) // hid == 2,
            jnp.float32(1.0), jnp.float32(0.5))
        for l in range(num_layers):
            wcat_ref[l * kk:l * kk + hid, :] = (
                whh_ref[l] * (0.5 * sc)).astype(jnp.bfloat16)
            wcat_ref[l * kk + hid:(l + 1) * kk, :] = (
                wih_ref[l] * (sc if l == 0 else 0.5 * sc)).astype(jnp.bfloat16)

        tc_ = min(8, seq_len)
        for t0 in range(0, seq_len, tc_):
            blk = x_ref[:, t0:t0 + tc_, :].astype(jnp.bfloat16)
            blk = jnp.transpose(blk, (1, 0, 2))
            seq_ref[t0 * b_pad:(t0 + tc_) * b_pad, :] = (
                blk.reshape(tc_ * b_pad, d_pad))

        for l in range(num_layers):
            b = b_ref[l] * sc
            h = [jnp.zeros((bs, hid), jnp.float32) for _ in range(n_streams)]
            c = [jnp.zeros((bs, hid), jnp.float32) for _ in range(n_streams)]
            hb = [None] * n_streams
            for t in range(seq_len):
                for s in range(n_streams):
                    r0 = t * b_pad + s * bs
                    s_t = seq_ref[r0:r0 + bs, :]
                    if t == 0:
                        g = jnp.dot(s_t, wcat_ref[l * kk + hid:(l + 1) * kk, :],
                                    preferred_element_type=jnp.float32) + b
                    else:
                        lhs = jnp.concatenate([hb[s], s_t], axis=1)
                        g = jnp.dot(lhs, wcat_ref[l * kk:(l + 1) * kk, :],
                                    preferred_element_type=jnp.float32) + b

                    ti = jnp.tanh(g[:, 0 * hid:1 * hid])
                    tf = jnp.tanh(g[:, 1 * hid:2 * hid])
                    tg = jnp.tanh(g[:, 2 * hid:3 * hid])
                    to = jnp.tanh(g[:, 3 * hid:4 * hid])

                    c[s] = 0.5 * ((tf * c[s] + c[s]) + (ti * tg + tg))
                    tcell = jnp.tanh(c[s])
                    h[s] = to * tcell + tcell
                    hb[s] = h[s].astype(jnp.bfloat16)
                    seq_ref[r0:r0 + bs, :hid] = hb[s]

            hout = jnp.concatenate(h, axis=0) if n_streams > 1 else h[0]
            hid_ref[l] = 0.5 * hout
            cell_ref[l] = jnp.concatenate(c, axis=0) if n_streams > 1 else c[0]

    return body


def kernel(x, w_ih_all, w_hh_all, b_all):
    num_layers, d_pad, four_h = w_ih_all.shape
    hid = four_h // 4
    B, T, D = x.shape

    n_streams = 2
    b_pad = max(8 * n_streams, -(-B // (8 * n_streams)) * (8 * n_streams))
    if b_pad != B or d_pad != D:
        x = jnp.pad(x, ((0, b_pad - B), (0, 0), (0, d_pad - D)))

    body = _make_lstm_body(num_layers, T, b_pad, d_pad, hid, n_streams)

    hidden, cell = pl.pallas_call(
        body,
        out_shape=(
            jax.ShapeDtypeStruct((num_layers, b_pad, hid), jnp.float32),
            jax.ShapeDtypeStruct((num_layers, b_pad, hid), jnp.float32),
        ),
        scratch_shapes=[
            pltpu.VMEM((T * b_pad, d_pad), jnp.bfloat16),
            pltpu.VMEM((num_layers * (hid + d_pad), four_h), jnp.bfloat16),
        ],
    )(x, w_hh_all, w_ih_all, b_all)

    if b_pad != B:
        hidden, cell = hidden[:, :B, :], cell[:, :B, :]
    return hidden, cell


# speedup vs baseline: 1.0176x; 1.0176x over previous
import jax
import jax.numpy as jnp
from jax.experimental import pallas as pl
from jax.experimental.pallas import tpu as pltpu


def _make_lstm_body(num_layers, seq_len, b_pad, d_pad, hid, n_streams):
    bs = b_pad // n_streams
    kk = hid + d_pad

    def body(x_ref, whh_ref, wih_ref, b_ref,
             hid_ref, cell_ref,
             seq_ref, wcat_ref):
        sc = jnp.where(
            jax.lax.broadcasted_iota(jnp.int32, (1, 4 * hid), 1) // hid == 2,
            jnp.float32(1.0), jnp.float32(0.5))
        for l in range(num_layers):
            wcat_ref[l * kk:l * kk + hid, :] = (
                whh_ref[l] * sc).astype(jnp.bfloat16)
            wcat_ref[l * kk + hid:(l + 1) * kk, :] = (
                wih_ref[l] * sc).astype(jnp.bfloat16)

        tc_ = min(8, seq_len)
        for t0 in range(0, seq_len, tc_):
            blk = x_ref[:, t0:t0 + tc_, :].astype(jnp.bfloat16)
            blk = jnp.transpose(blk, (1, 0, 2))
            seq_ref[t0 * b_pad:(t0 + tc_) * b_pad, :] = (
                blk.reshape(tc_ * b_pad, d_pad))

        for l in range(num_layers):
            b = b_ref[l] * sc
            h = [jnp.zeros((bs, hid), jnp.float32) for _ in range(n_streams)]
            c = [jnp.zeros((bs, hid), jnp.float32) for _ in range(n_streams)]
            hb = [None] * n_streams
            for t in range(seq_len):
                for s in range(n_streams):
                    r0 = t * b_pad + s * bs
                    s_t = seq_ref[r0:r0 + bs, :]
                    if t == 0:
                        g = jnp.dot(s_t, wcat_ref[l * kk + hid:(l + 1) * kk, :],
                                    preferred_element_type=jnp.float32) + b
                    else:
                        lhs = jnp.concatenate([hb[s], s_t], axis=1)
                        g = jnp.dot(lhs, wcat_ref[l * kk:(l + 1) * kk, :],
                                    preferred_element_type=jnp.float32) + b

                    ti = jnp.tanh(g[:, 0 * hid:1 * hid])
                    tf = jnp.tanh(g[:, 1 * hid:2 * hid])
                    tg = jnp.tanh(g[:, 2 * hid:3 * hid])
                    to = jnp.tanh(g[:, 3 * hid:4 * hid])

                    c[s] = 0.5 * ((tf * c[s] + c[s]) + (ti * tg + tg))
                    tcell = jnp.tanh(c[s])
                    h[s] = 0.5 * (to * tcell + tcell)
                    hb[s] = h[s].astype(jnp.bfloat16)
                    seq_ref[r0:r0 + bs, :hid] = hb[s]

            hid_ref[l] = jnp.concatenate(h, axis=0) if n_streams > 1 else h[0]
            cell_ref[l] = jnp.concatenate(c, axis=0) if n_streams > 1 else c[0]

    return body


def kernel(x, w_ih_all, w_hh_all, b_all):
    num_layers, d_pad, four_h = w_ih_all.shape
    hid = four_h // 4
    B, T, D = x.shape

    n_streams = 2
    b_pad = max(8 * n_streams, -(-B // (8 * n_streams)) * (8 * n_streams))
    if b_pad != B or d_pad != D:
        x = jnp.pad(x, ((0, b_pad - B), (0, 0), (0, d_pad - D)))

    body = _make_lstm_body(num_layers, T, b_pad, d_pad, hid, n_streams)

    hidden, cell = pl.pallas_call(
        body,
        out_shape=(
            jax.ShapeDtypeStruct((num_layers, b_pad, hid), jnp.float32),
            jax.ShapeDtypeStruct((num_layers, b_pad, hid), jnp.float32),
        ),
        scratch_shapes=[
            pltpu.VMEM((T * b_pad, d_pad), jnp.bfloat16),
            pltpu.VMEM((num_layers * (hid + d_pad), four_h), jnp.bfloat16),
        ],
    )(x, w_hh_all, w_ih_all, b_all)

    if b_pad != B:
        hidden, cell = hidden[:, :B, :], cell[:, :B, :]
    return hidden, cell
